# baseline (speedup 1.0000x reference)
import functools

import jax
import jax.numpy as jnp
from jax import lax
from jax.experimental import pallas as pl
from jax.experimental.pallas import tpu as pltpu
from jax.experimental.pallas import tpu_sc as plsc

_NUM_WORKERS = 32
_NBUF = 4
_CH = 8
_LAG = 2


def _gather_kernel(batch, seq, d, ids_hbm, table_hbm, out_hbm,
                   idx_v, buf, gsem, ssem):
    n_rows = batch * seq
    b_per_w = n_rows // _NUM_WORKERS
    n_chunks = b_per_w // _CH
    per_row = seq // b_per_w
    wid = lax.axis_index("s") * 2 + lax.axis_index("c")
    row = wid // per_row
    col = (wid % per_row) * b_per_w
    pltpu.sync_copy(ids_hbm.at[row, pl.ds(col, b_per_w)], idx_v)

    def slot(i):
        off = lax.rem(i, _NBUF) * _CH
        return buf.at[pl.ds(pl.multiple_of(off, _CH), _CH)]

    def gather(i):
        return pltpu.make_async_copy(
            table_hbm.at[idx_v.at[pl.ds(pl.multiple_of(i * _CH, _CH), _CH)]],
            slot(i), gsem,
        )

    def scatter(i):
        return pltpu.make_async_copy(
            slot(i),
            out_hbm.at[row, pl.ds(pl.multiple_of(col + i * _CH, _CH), _CH)],
            ssem,
        )

    for b2 in range(_NBUF):
        gather(b2).start()

    def step(i, carry):
        gather(i).wait()
        scatter(i).start()

        @pl.when(jnp.logical_and(i >= _LAG, i - _LAG + _NBUF < n_chunks))
        def _():
            scatter(i - _LAG).wait()
            gather(i - _LAG + _NBUF).start()

        return carry

    lax.fori_loop(0, n_chunks, step, 0)
    for _ in range(_NBUF):
        scatter(0).wait()


def kernel(input_ids, embed_table):
    b, s = input_ids.shape
    v, d = embed_table.shape

    mesh = plsc.VectorSubcoreMesh(core_axis_name="c", subcore_axis_name="s")
    run = pl.kernel(
        functools.partial(_gather_kernel, b, s, d),
        mesh=mesh,
        out_type=jax.ShapeDtypeStruct((b, s, d), jnp.float32),
        scratch_types=[
            pltpu.VMEM((b * s // _NUM_WORKERS,), jnp.int32),
            pltpu.VMEM((_NBUF * _CH, d), jnp.float32),
            pltpu.SemaphoreType.DMA,
            pltpu.SemaphoreType.DMA,
        ],
    )
    return run(input_ids, embed_table)


# speedup vs baseline: 1.0167x; 1.0167x over previous
import functools

import jax
import jax.numpy as jnp
from jax import lax
from jax.experimental import pallas as pl
from jax.experimental.pallas import tpu as pltpu
from jax.experimental.pallas import tpu_sc as plsc

_NUM_WORKERS = 32
_NBUF = 4
_CH = 8


def _gather_kernel(batch, seq, d, ids_hbm, table_hbm, out_hbm,
                   idx_v, buf, region, gsem, csem, ssem, dsem):
    n_rows = batch * seq
    b_per_w = n_rows // _NUM_WORKERS
    n_chunks = b_per_w // _CH
    n_groups = n_chunks // _NBUF
    per_row = seq // b_per_w
    s = lax.axis_index("s")
    wid = s * 2 + lax.axis_index("c")
    row = wid // per_row
    col = (wid % per_row) * b_per_w
    pltpu.sync_copy(ids_hbm.at[row, pl.ds(col, b_per_w)], idx_v)
    myregion = region.at[s]

    def gather(i, b2):
        return pltpu.make_async_copy(
            table_hbm.at[idx_v.at[pl.ds(i * _CH, _CH)]], buf.at[b2], gsem
        )

    def direct(i):
        return pltpu.make_async_copy(
            buf.at[0], out_hbm.at[row, pl.ds(col + i * _CH, _CH)], dsem
        )

    def tospmem(b2):
        return pltpu.make_async_copy(buf.at[b2], myregion.at[b2 - 1], csem)

    def drain(i, b2):
        return pltpu.make_async_copy(
            myregion.at[b2 - 1], out_hbm.at[row, pl.ds(col + i * _CH, _CH)], ssem
        )

    for b2 in range(_NBUF):
        gather(b2, b2).start()

    def grp(g, carry):
        for b2 in range(_NBUF):
            i = g * _NBUF + b2
            gather(i, b2).wait()
            if b2 == 0:
                direct(i).start()

                @pl.when(g < n_groups - 1)
                def _():
                    direct(i).wait()
                    gather(i + _NBUF, 0).start()
            else:

                @pl.when(g > 0)
                def _():
                    drain(i, b2).wait()

                tospmem(b2).start()
                tospmem(b2).wait()
                drain(i, b2).start()

                @pl.when(g < n_groups - 1)
                def _():
                    gather(i + _NBUF, b2).start()

        return carry

    lax.fori_loop(0, n_groups, grp, 0)
    direct(0).wait()
    for b2 in range(1, _NBUF):
        drain(0, b2).wait()


def kernel(input_ids, embed_table):
    b, s = input_ids.shape
    v, d = embed_table.shape

    mesh = plsc.VectorSubcoreMesh(core_axis_name="c", subcore_axis_name="s")
    run = pl.kernel(
        functools.partial(_gather_kernel, b, s, d),
        mesh=mesh,
        out_type=jax.ShapeDtypeStruct((b, s, d), jnp.float32),
        scratch_types=[
            pltpu.VMEM((b * s // _NUM_WORKERS,), jnp.int32),
            pltpu.VMEM((_NBUF, _CH, d), jnp.float32),
            pltpu.MemorySpace.VMEM_SHARED((16, _NBUF - 1, _CH, d), jnp.float32),
            pltpu.SemaphoreType.DMA,
            pltpu.SemaphoreType.DMA,
            pltpu.SemaphoreType.DMA,
            pltpu.SemaphoreType.DMA,
        ],
    )
    return run(input_ids, embed_table)
